# Initial kernel scaffold; baseline (speedup 1.0000x reference)
import jax
import jax.numpy as jnp
from jax.experimental import pallas as pl


def kernel(x, edge_index, edge_attr, W1, b1, W2, b2, Wl, bl, Wn1, bn1, Wn2, bn2, Wo, bo):
    raise NotImplementedError("write your pallas kernel here")


# speedup vs baseline: 2.1070x; 2.1070x over previous
import functools

import jax
import jax.numpy as jnp
from jax import lax
from jax.experimental import pallas as pl
from jax.experimental.pallas import tpu as pltpu
from jax.experimental.pallas import tpu_sc as plsc

H = 64
HH = 32
L = 16
NC = 2
NS = 16
CHUNK = 256
SUB = 128
CPB = CHUNK // SUB
ZB = 224

BE = 4096
BN = 8192


def _edge_body(ea_ref, w1_ref, b1_ref, w2_ref, b2_ref, wl_ref, bl_ref, out_ref):
    w2l = jnp.dot(w2_ref[...], wl_ref[...], preferred_element_type=jnp.float32)
    b2l = (jnp.dot(b2_ref[...], wl_ref[...], preferred_element_type=jnp.float32)
           + bl_ref[...])
    t = jnp.maximum(
        jnp.dot(ea_ref[...], w1_ref[...], preferred_element_type=jnp.float32)
        + b1_ref[...], 0.0)
    e = jnp.dot(t, w2l, preferred_element_type=jnp.float32) + b2l
    out_ref[0] = e[:, :HH]
    out_ref[1] = e[:, HH:]


def _node_body(x_ref, a_ref, wn1_ref, bn1_ref, wn2_ref, bn2_ref, wo_ref,
               bo_ref, out_ref):
    h = x_ref[...] + jnp.concatenate([a_ref[0], a_ref[1]], axis=1)
    t = jnp.maximum(
        jnp.dot(h, wn1_ref[...], preferred_element_type=jnp.float32)
        + bn1_ref[...], 0.0)
    w2o = jnp.dot(wn2_ref[...], wo_ref[...], preferred_element_type=jnp.float32)
    b2o = (jnp.dot(bn2_ref[...], wo_ref[...], preferred_element_type=jnp.float32)
           + bo_ref[...])
    out_ref[...] = jnp.dot(t, w2o, preferred_element_type=jnp.float32) + b2o


def _make_sc_body(n_nodes, e_pad, n_acc):
    ep_tile = e_pad // NS
    kchunks = ep_tile // CHUNK
    rows_per_tile = n_acc // NS
    zb = ZB

    def sc_body(x2f, srcp2, dstp2, e2f, outf, idx_s, idx_d, rows_v, e_v, acc,
                sem):
        c = lax.axis_index("c")
        s = lax.axis_index("s")

        z = jnp.zeros((L,), jnp.float32)

        def zrow(r, _):
            rows_v[r, pl.ds(0, L)] = z
            rows_v[r, pl.ds(L, L)] = z
            return 0

        lax.fori_loop(0, zb, zrow, 0)
        for j in range(rows_per_tile // zb):
            pltpu.sync_copy(rows_v.at[pl.ds(0, zb)],
                            acc.at[pl.ds(s * rows_per_tile + j * zb, zb)])
        plsc.subcore_barrier()

        cN = c * n_nodes

        def step(k, _):
            base = s * ep_tile + k * CHUNK
            rbase = s * (ep_tile // SUB) + k * CPB
            pltpu.sync_copy(srcp2.at[pl.ds(rbase, CPB)], idx_s)
            pltpu.sync_copy(dstp2.at[pl.ds(rbase, CPB)], idx_d)
            pltpu.sync_copy(e2f.at[pl.ds(c * e_pad + base, CHUNK)], e_v)

            def adj(r, _):
                for t in range(SUB // L):
                    sl = pl.ds(t * L, L)
                    idx_s[r, sl] = idx_s[r, sl] + cN
                return 0

            lax.fori_loop(0, CPB, adj, 0)

            cps = [pltpu.async_copy(x2f.at[idx_s.at[j]],
                                    rows_v.at[pl.ds(j * SUB, SUB)], sem)
                   for j in range(CPB)]
            for cp in cps:
                cp.wait()

            def crow(r, _):
                for u in range(4):
                    ri = r * 4 + u
                    for t in range(2):
                        sl = pl.ds(t * L, L)
                        rows_v[ri, sl] = jnp.maximum(
                            rows_v[ri, sl] + e_v[ri, sl], 0.0)
                return 0

            lax.fori_loop(0, CHUNK // 4, crow, 0)

            for j in range(CPB):
                pltpu.sync_copy(rows_v.at[pl.ds(j * SUB, SUB)],
                                acc.at[idx_d.at[j]], add=True)
            return 0

        lax.fori_loop(0, kchunks, step, 0)
        plsc.subcore_barrier()

        for j in range(rows_per_tile // zb):
            off = s * rows_per_tile + j * zb
            pltpu.sync_copy(acc.at[pl.ds(off, zb)], rows_v.at[pl.ds(0, zb)])
            pltpu.sync_copy(rows_v.at[pl.ds(0, zb)],
                            outf.at[pl.ds(c * n_acc + off, zb)])

    return sc_body


def kernel(x, edge_index, edge_attr, W1, b1, W2, b2, Wl, bl, Wn1, bn1, Wn2,
           bn2, Wo, bo):
    n, h = x.shape
    e_edges = edge_index.shape[1]
    d_edge = edge_attr.shape[1]
    assert h == H

    e_pad = ((e_edges + NS * CHUNK - 1) // (NS * CHUNK)) * (NS * CHUNK)
    n_acc = ((n + 1 + NS * ZB - 1) // (NS * ZB)) * (NS * ZB)
    pad = e_pad - e_edges

    src = edge_index[0].astype(jnp.int32)
    dst = edge_index[1].astype(jnp.int32)
    src_p = jnp.concatenate([src, jnp.zeros((pad,), jnp.int32)])
    dst_p = jnp.concatenate([dst, jnp.full((pad,), n, jnp.int32)])
    srcp2 = src_p.reshape(e_pad // SUB, SUB)
    dstp2 = dst_p.reshape(e_pad // SUB, SUB)
    ea_p = jnp.concatenate(
        [edge_attr, jnp.zeros((pad, d_edge), edge_attr.dtype)])

    e2 = pl.pallas_call(
        _edge_body,
        grid=(e_pad // BE,),
        in_specs=[
            pl.BlockSpec((BE, d_edge), lambda i: (i, 0)),
            pl.BlockSpec((d_edge, H), lambda i: (0, 0)),
            pl.BlockSpec((1, H), lambda i: (0, 0)),
            pl.BlockSpec((H, H), lambda i: (0, 0)),
            pl.BlockSpec((1, H), lambda i: (0, 0)),
            pl.BlockSpec((H, H), lambda i: (0, 0)),
            pl.BlockSpec((1, H), lambda i: (0, 0)),
        ],
        out_specs=pl.BlockSpec((2, BE, HH), lambda i: (0, i, 0)),
        out_shape=jax.ShapeDtypeStruct((2, e_pad, HH), jnp.float32),
    )(ea_p, W1, b1.reshape(1, H), W2, b2.reshape(1, H), Wl, bl.reshape(1, H))

    x2f = jnp.concatenate([x[:, :HH], x[:, HH:]], axis=0)
    e2f = e2.reshape(2 * e_pad, HH)

    sc = pl.kernel(
        _make_sc_body(n, e_pad, n_acc),
        out_type=jax.ShapeDtypeStruct((2 * n_acc, HH), jnp.float32),
        mesh=plsc.VectorSubcoreMesh(core_axis_name="c", subcore_axis_name="s",
                                    num_cores=NC, num_subcores=NS),
        scratch_types=[
            pltpu.VMEM((CPB, SUB), jnp.int32),
            pltpu.VMEM((CPB, SUB), jnp.int32),
            pltpu.VMEM((CHUNK, HH), jnp.float32),
            pltpu.VMEM((CHUNK, HH), jnp.float32),
            pltpu.VMEM_SHARED((n_acc, HH), jnp.float32),
            pltpu.SemaphoreType.DMA,
        ],
        compiler_params=pltpu.CompilerParams(use_tc_tiling_on_sc=False),
    )
    aggr2 = sc(x2f, srcp2, dstp2, e2f).reshape(2, n_acc, HH)

    out = pl.pallas_call(
        _node_body,
        grid=(pl.cdiv(n, BN),),
        in_specs=[
            pl.BlockSpec((BN, H), lambda i: (i, 0)),
            pl.BlockSpec((2, BN, HH), lambda i: (0, i, 0)),
            pl.BlockSpec((H, H), lambda i: (0, 0)),
            pl.BlockSpec((1, H), lambda i: (0, 0)),
            pl.BlockSpec((H, H), lambda i: (0, 0)),
            pl.BlockSpec((1, H), lambda i: (0, 0)),
            pl.BlockSpec((H, 3), lambda i: (0, 0)),
            pl.BlockSpec((1, 3), lambda i: (0, 0)),
        ],
        out_specs=pl.BlockSpec((BN, 3), lambda i: (i, 0)),
        out_shape=jax.ShapeDtypeStruct((n, 3), jnp.float32),
    )(x, aggr2, Wn1, bn1.reshape(1, H), Wn2, bn2.reshape(1, H), Wo,
      bo.reshape(1, 3))

    return out.reshape(-1, 9, 3)


# speedup vs baseline: 2.6700x; 1.2672x over previous
import functools

import jax
import jax.numpy as jnp
from jax import lax
from jax.experimental import pallas as pl
from jax.experimental.pallas import tpu as pltpu
from jax.experimental.pallas import tpu_sc as plsc

H = 64
HH = 32
L = 16
NC = 2
NS = 16
CHUNK = 256
SUB = 128
CPB = CHUNK // SUB
ZB = 224

BE = 4096
BN = 8192


def _edge_body(ea_ref, w1_ref, b1_ref, w2_ref, b2_ref, wl_ref, bl_ref, out_ref):
    w2l = jnp.dot(w2_ref[...], wl_ref[...], preferred_element_type=jnp.float32)
    b2l = (jnp.dot(b2_ref[...], wl_ref[...], preferred_element_type=jnp.float32)
           + bl_ref[...])
    t = jnp.maximum(
        jnp.dot(ea_ref[...], w1_ref[...], preferred_element_type=jnp.float32)
        + b1_ref[...], 0.0)
    e = jnp.dot(t, w2l, preferred_element_type=jnp.float32) + b2l
    q = BE // 4
    out_ref[0] = jnp.concatenate(
        [e[i * q:(i + 1) * q, :HH] for i in range(4)], axis=1)
    out_ref[1] = jnp.concatenate(
        [e[i * q:(i + 1) * q, HH:] for i in range(4)], axis=1)


def _node_body(x_ref, a_ref, wn1_ref, bn1_ref, wn2_ref, bn2_ref, wo_ref,
               bo_ref, out_ref):
    h = x_ref[...] + jnp.concatenate([a_ref[0], a_ref[1]], axis=1)
    t = jnp.maximum(
        jnp.dot(h, wn1_ref[...], preferred_element_type=jnp.float32)
        + bn1_ref[...], 0.0)
    w2o = jnp.dot(wn2_ref[...], wo_ref[...], preferred_element_type=jnp.float32)
    b2o = (jnp.dot(bn2_ref[...], wo_ref[...], preferred_element_type=jnp.float32)
           + bo_ref[...])
    out_ref[...] = jnp.dot(t, w2o, preferred_element_type=jnp.float32) + b2o


def _make_sc_body(n_nodes, e_pad, n_acc):
    ep_tile = e_pad // NS
    kchunks = ep_tile // CHUNK
    rows_per_tile = n_acc // NS
    zb = ZB

    def sc_body(x2f, srcp2, dstp2, e2f, outf, idx_s, idx_d, rows_v, e_v, acc,
                sem):
        c = lax.axis_index("c")
        s = lax.axis_index("s")

        z = jnp.zeros((L,), jnp.float32)

        def zrow(r, _):
            rows_v[r, pl.ds(0, L)] = z
            rows_v[r, pl.ds(L, L)] = z
            return 0

        lax.fori_loop(0, zb, zrow, 0)
        for j in range(rows_per_tile // zb):
            pltpu.sync_copy(rows_v.at[pl.ds(0, zb)],
                            acc.at[pl.ds(s * rows_per_tile + j * zb, zb)])
        plsc.subcore_barrier()

        cN = c * n_nodes

        def step(k, _):
            base = s * ep_tile + k * CHUNK
            rbase = s * (ep_tile // SUB) + k * CPB
            pltpu.sync_copy(srcp2.at[pl.ds(rbase, CPB)], idx_s)
            pltpu.sync_copy(dstp2.at[pl.ds(rbase, CPB)], idx_d)
            blk = base // BE
            o = base - blk * BE
            g = o // (BE // 4)
            ro = o - g * (BE // 4)
            r0 = blk * (BE // 4) + ro
            pltpu.sync_copy(
                e2f.at[pl.ds(c * (e_pad // 4) + r0, CHUNK), pl.ds(g * HH, HH)],
                e_v)

            def adj(r, _):
                for t in range(SUB // L):
                    sl = pl.ds(t * L, L)
                    idx_s[r, sl] = idx_s[r, sl] + cN
                return 0

            lax.fori_loop(0, CPB, adj, 0)

            cps = [pltpu.async_copy(x2f.at[idx_s.at[j]],
                                    rows_v.at[pl.ds(j * SUB, SUB)], sem)
                   for j in range(CPB)]
            for cp in cps:
                cp.wait()

            def crow(r, _):
                for u in range(4):
                    ri = r * 4 + u
                    for t in range(2):
                        sl = pl.ds(t * L, L)
                        rows_v[ri, sl] = jnp.maximum(
                            rows_v[ri, sl] + e_v[ri, sl], 0.0)
                return 0

            lax.fori_loop(0, CHUNK // 4, crow, 0)

            for j in range(CPB):
                pltpu.sync_copy(rows_v.at[pl.ds(j * SUB, SUB)],
                                acc.at[idx_d.at[j]], add=True)
            return 0

        lax.fori_loop(0, kchunks, step, 0)
        plsc.subcore_barrier()

        for j in range(rows_per_tile // zb):
            off = s * rows_per_tile + j * zb
            pltpu.sync_copy(acc.at[pl.ds(off, zb)], rows_v.at[pl.ds(0, zb)])
            pltpu.sync_copy(rows_v.at[pl.ds(0, zb)],
                            outf.at[pl.ds(c * n_acc + off, zb)])

    return sc_body


def kernel(x, edge_index, edge_attr, W1, b1, W2, b2, Wl, bl, Wn1, bn1, Wn2,
           bn2, Wo, bo):
    n, h = x.shape
    e_edges = edge_index.shape[1]
    d_edge = edge_attr.shape[1]
    assert h == H

    e_pad = ((e_edges + NS * CHUNK - 1) // (NS * CHUNK)) * (NS * CHUNK)
    n_acc = ((n + 1 + NS * ZB - 1) // (NS * ZB)) * (NS * ZB)
    pad = e_pad - e_edges

    src = edge_index[0].astype(jnp.int32)
    dst = edge_index[1].astype(jnp.int32)
    src_p = jnp.concatenate([src, jnp.zeros((pad,), jnp.int32)])
    dst_p = jnp.concatenate([dst, jnp.full((pad,), n, jnp.int32)])
    srcp2 = src_p.reshape(e_pad // SUB, SUB)
    dstp2 = dst_p.reshape(e_pad // SUB, SUB)
    ea_p = jnp.concatenate(
        [edge_attr, jnp.zeros((pad, d_edge), edge_attr.dtype)])

    e2 = pl.pallas_call(
        _edge_body,
        grid=(e_pad // BE,),
        in_specs=[
            pl.BlockSpec((BE, d_edge), lambda i: (i, 0)),
            pl.BlockSpec((d_edge, H), lambda i: (0, 0)),
            pl.BlockSpec((1, H), lambda i: (0, 0)),
            pl.BlockSpec((H, H), lambda i: (0, 0)),
            pl.BlockSpec((1, H), lambda i: (0, 0)),
            pl.BlockSpec((H, H), lambda i: (0, 0)),
            pl.BlockSpec((1, H), lambda i: (0, 0)),
        ],
        out_specs=pl.BlockSpec((2, BE // 4, 4 * HH), lambda i: (0, i, 0)),
        out_shape=jax.ShapeDtypeStruct((2, e_pad // 4, 4 * HH), jnp.float32),
    )(ea_p, W1, b1.reshape(1, H), W2, b2.reshape(1, H), Wl, bl.reshape(1, H))

    x2f = jnp.concatenate([x[:, :HH], x[:, HH:]], axis=0)
    e2f = e2.reshape(2 * (e_pad // 4), 4 * HH)

    sc = pl.kernel(
        _make_sc_body(n, e_pad, n_acc),
        out_type=jax.ShapeDtypeStruct((2 * n_acc, HH), jnp.float32),
        mesh=plsc.VectorSubcoreMesh(core_axis_name="c", subcore_axis_name="s",
                                    num_cores=NC, num_subcores=NS),
        scratch_types=[
            pltpu.VMEM((CPB, SUB), jnp.int32),
            pltpu.VMEM((CPB, SUB), jnp.int32),
            pltpu.VMEM((CHUNK, HH), jnp.float32),
            pltpu.VMEM((CHUNK, HH), jnp.float32),
            pltpu.VMEM_SHARED((n_acc, HH), jnp.float32),
            pltpu.SemaphoreType.DMA,
        ],
        compiler_params=pltpu.CompilerParams(use_tc_tiling_on_sc=False),
    )
    aggr2 = sc(x2f, srcp2, dstp2, e2f).reshape(2, n_acc, HH)

    out = pl.pallas_call(
        _node_body,
        grid=(pl.cdiv(n, BN),),
        in_specs=[
            pl.BlockSpec((BN, H), lambda i: (i, 0)),
            pl.BlockSpec((2, BN, HH), lambda i: (0, i, 0)),
            pl.BlockSpec((H, H), lambda i: (0, 0)),
            pl.BlockSpec((1, H), lambda i: (0, 0)),
            pl.BlockSpec((H, H), lambda i: (0, 0)),
            pl.BlockSpec((1, H), lambda i: (0, 0)),
            pl.BlockSpec((H, 3), lambda i: (0, 0)),
            pl.BlockSpec((1, 3), lambda i: (0, 0)),
        ],
        out_specs=pl.BlockSpec((BN, 3), lambda i: (i, 0)),
        out_shape=jax.ShapeDtypeStruct((n, 3), jnp.float32),
    )(x, aggr2, Wn1, bn1.reshape(1, H), Wn2, bn2.reshape(1, H), Wo,
      bo.reshape(1, 3))

    return out.reshape(-1, 9, 3)


# speedup vs baseline: 7.6737x; 2.8740x over previous
import functools
import math

import jax
import jax.numpy as jnp
from jax import lax
from jax.experimental import pallas as pl
from jax.experimental.pallas import tpu as pltpu
from jax.experimental.pallas import tpu_sc as plsc

H = 64
HH = 32
L = 16
NC = 2
NS = 16
CHUNK = 128
SUB = 128
ZB = 184

BE = 4096
BN = 6400


def _edge_body(ea_ref, w1_ref, b1_ref, w2_ref, b2_ref, wl_ref, bl_ref, out_ref):
    w2l = jnp.dot(w2_ref[...], wl_ref[...], preferred_element_type=jnp.float32)
    b2l = (jnp.dot(b2_ref[...], wl_ref[...], preferred_element_type=jnp.float32)
           + bl_ref[...])
    t = jnp.maximum(
        lax.dot_general(ea_ref[...], w1_ref[...], (((0,), (0,)), ((), ())),
                        preferred_element_type=jnp.float32)
        + b1_ref[...], 0.0)
    e = jnp.dot(t, w2l, preferred_element_type=jnp.float32) + b2l
    q = BE // 4
    out_ref[0] = jnp.concatenate(
        [e[i * q:(i + 1) * q, :HH] for i in range(4)], axis=1)
    out_ref[1] = jnp.concatenate(
        [e[i * q:(i + 1) * q, HH:] for i in range(4)], axis=1)


def _node_body(xt_ref, alo_ref, ahi_ref, blo_ref, bhi_ref, wn1_ref, bn1_ref,
               wn2_ref, bn2_ref, wo_ref, bo_ref, out_ref):
    wn1 = wn1_ref[...]
    hw = lax.dot_general(xt_ref[...], wn1, (((0,), (0,)), ((), ())),
                         preferred_element_type=jnp.float32)
    hw = hw + jnp.dot(alo_ref[...] + blo_ref[...], wn1[:HH],
                      preferred_element_type=jnp.float32)
    hw = hw + jnp.dot(ahi_ref[...] + bhi_ref[...], wn1[HH:],
                      preferred_element_type=jnp.float32)
    t = jnp.maximum(hw + bn1_ref[...], 0.0)
    w2o = jnp.dot(wn2_ref[...], wo_ref[...], preferred_element_type=jnp.float32)
    b2o = (jnp.dot(bn2_ref[...], wo_ref[...], preferred_element_type=jnp.float32)
           + bo_ref[...])
    out_ref[...] = jnp.dot(t, w2o, preferred_element_type=jnp.float32) + b2o


def _make_sc_body(n_nodes, e_pad, n_acc):
    ep_tile = e_pad // NS
    kchunks = ep_tile // CHUNK
    rows_per_tile = n_acc // NS
    assert kchunks % 3 == 0 and kchunks >= 6
    assert rows_per_tile % SUB == 0

    def sc_body(x2f, srcp2, dstp2, e2f, outf,
                src_i, dst_i, rows_v, e_v, acc,
                lsem0, lsem1, lsem2, gsem0, gsem1, gsem2,
                ssem0, ssem1, ssem2):
        c = lax.axis_index("c")
        s = lax.axis_index("s")
        lsem = (lsem0, lsem1, lsem2)
        gsem = (gsem0, gsem1, gsem2)
        ssem = (ssem0, ssem1, ssem2)

        z = jnp.zeros((L,), jnp.float32)

        def zrow(r, _):
            rows_v[0, r, pl.ds(0, L)] = z
            rows_v[0, r, pl.ds(L, L)] = z
            return 0

        lax.fori_loop(0, SUB, zrow, 0)
        for j in range(rows_per_tile // SUB):
            pltpu.sync_copy(rows_v.at[0],
                            acc.at[pl.ds(s * rows_per_tile + j * SUB, SUB)])
        plsc.subcore_barrier()

        def lrow(k):
            return k * NS + s

        def srow(k):
            return c * (e_pad // SUB) + lrow(k)

        def eslice(k):
            base = lrow(k) * CHUNK
            blk = base // BE
            o = base - blk * BE
            g = o // (BE // 4)
            ro = o - g * (BE // 4)
            r0 = blk * (BE // 4) + ro
            return e2f.at[c, pl.ds(r0, CHUNK), pl.ds(g * HH, HH)]

        def issue_load(k, j3, sem):
            j5 = jnp.remainder(k, 5)
            pltpu.async_copy(srcp2.at[srow(k)], src_i.at[j3], sem)
            pltpu.async_copy(dstp2.at[lrow(k)], dst_i.at[j5], sem)
            pltpu.async_copy(eslice(k), e_v.at[j3], sem)

        def wait_load(k, j3, sem):
            j5 = jnp.remainder(k, 5)
            pltpu.make_async_copy(srcp2.at[srow(k)], src_i.at[j3], sem).wait()
            pltpu.make_async_copy(dstp2.at[lrow(k)], dst_i.at[j5], sem).wait()
            pltpu.make_async_copy(eslice(k), e_v.at[j3], sem).wait()

        def issue_gather(j3, sem):
            pltpu.async_copy(x2f.at[src_i.at[j3]], rows_v.at[j3], sem)

        def wait_gather(j3, sem):
            pltpu.make_async_copy(x2f.at[src_i.at[j3]], rows_v.at[j3],
                                  sem).wait()

        def compute(j3):
            def crow(r, _):
                for u in range(4):
                    ri = r * 4 + u
                    for t in range(2):
                        sl = pl.ds(t * L, L)
                        rows_v[j3, ri, sl] = jnp.maximum(
                            rows_v[j3, ri, sl] + e_v[j3, ri, sl], 0.0)
                return 0

            lax.fori_loop(0, SUB // 4, crow, 0)

        def issue_scatter(k, j3, sem):
            j5 = jnp.remainder(k, 5)
            pltpu.async_copy(rows_v.at[j3], acc.at[dst_i.at[j5]], sem,
                             add=True)

        def wait_scatter(k, j3, sem):
            j5 = jnp.remainder(k, 5)
            pltpu.make_async_copy(rows_v.at[j3], acc.at[dst_i.at[j5]],
                                  sem).wait()

        pltpu.sync_copy(srcp2.at[srow(0)], src_i.at[0])
        pltpu.sync_copy(dstp2.at[lrow(0)], dst_i.at[0])
        pltpu.sync_copy(eslice(0), e_v.at[0])
        issue_gather(0, gsem[0])
        issue_load(1, 1, lsem[1])
        issue_load(2, 2, lsem[2])
        wait_load(1, 1, lsem[1])
        issue_gather(1, gsem[1])
        wait_gather(0, gsem[0])
        compute(0)
        issue_scatter(0, 0, ssem[0])
        issue_load(3, 0, lsem[0])
        wait_load(2, 2, lsem[2])
        issue_gather(2, gsem[2])
        wait_gather(1, gsem[1])
        compute(1)
        issue_scatter(1, 1, ssem[1])
        issue_load(4, 1, lsem[1])

        def body(k3, _):
            for par in range(3):
                k = 3 + 3 * k3 + par
                j3 = par
                jm1 = (par + 2) % 3
                wait_load(k, j3, lsem[j3])
                wait_scatter(k - 3, j3, ssem[j3])
                issue_gather(j3, gsem[j3])
                wait_gather(jm1, gsem[jm1])
                compute(jm1)
                issue_scatter(k - 1, jm1, ssem[jm1])
                issue_load(jnp.minimum(k + 2, kchunks - 1), jm1, lsem[jm1])
            return 0

        lax.fori_loop(0, (kchunks - 3) // 3, body, 0)

        wait_load(kchunks - 1, 0, lsem[0])
        wait_load(kchunks - 1, 1, lsem[1])
        wait_gather(2, gsem[2])
        compute(2)
        issue_scatter(kchunks - 1, 2, ssem[2])
        wait_scatter(kchunks - 3, 0, ssem[0])
        wait_scatter(kchunks - 2, 1, ssem[1])
        wait_scatter(kchunks - 1, 2, ssem[2])
        plsc.subcore_barrier()

        for j in range(rows_per_tile // SUB):
            off = s * rows_per_tile + j * SUB
            pltpu.sync_copy(acc.at[pl.ds(off, SUB)], rows_v.at[0])
            pltpu.sync_copy(rows_v.at[0], outf.at[pl.ds(c * n_acc + off, SUB)])

    return sc_body


def kernel(x, edge_index, edge_attr, W1, b1, W2, b2, Wl, bl, Wn1, bn1, Wn2,
           bn2, Wo, bo):
    n, h = x.shape
    e_edges = edge_index.shape[1]
    d_edge = edge_attr.shape[1]
    assert h == H

    ep_q = 3 * NS * CHUNK
    ep_q = ep_q * BE // math.gcd(ep_q, BE)
    n_half = (e_edges + 1) // 2
    e_half = ((n_half + ep_q - 1) // ep_q) * ep_q
    e_pad = 2 * e_half
    n_acc = ((n + 1 + NS * SUB - 1) // (NS * SUB)) * (NS * SUB)
    pad = e_pad - e_edges

    src = edge_index[0].astype(jnp.int32)
    dst = edge_index[1].astype(jnp.int32)
    src_p = jnp.concatenate([src, jnp.zeros((pad,), jnp.int32)])
    dst_p = jnp.concatenate([dst, jnp.full((pad,), n, jnp.int32)])
    ea_p = jnp.pad(edge_attr.T, ((0, 0), (0, pad)))
    x2f = jnp.concatenate([x[:, :HH], x[:, HH:]], axis=0)

    sc = pl.kernel(
        _make_sc_body(n, e_half, n_acc),
        out_type=jax.ShapeDtypeStruct((2 * n_acc, HH), jnp.float32),
        mesh=plsc.VectorSubcoreMesh(core_axis_name="c", subcore_axis_name="s",
                                    num_cores=NC, num_subcores=NS),
        scratch_types=[
            pltpu.VMEM((3, SUB), jnp.int32),
            pltpu.VMEM((5, SUB), jnp.int32),
            pltpu.VMEM((3, SUB, HH), jnp.float32),
            pltpu.VMEM((3, CHUNK, HH), jnp.float32),
            pltpu.VMEM_SHARED((n_acc, HH), jnp.float32),
        ] + [pltpu.SemaphoreType.DMA] * 9,
        compiler_params=pltpu.CompilerParams(use_tc_tiling_on_sc=False),
    )

    aggrs = []
    for hf in range(2):
        sl = slice(hf * e_half, (hf + 1) * e_half)
        srcp2 = jnp.concatenate(
            [src_p[sl], src_p[sl] + n]).reshape(2 * e_half // SUB, SUB)
        dstp2 = dst_p[sl].reshape(e_half // SUB, SUB)

        e2 = pl.pallas_call(
            _edge_body,
            grid=(e_half // BE,),
            in_specs=[
                pl.BlockSpec((d_edge, BE),
                             lambda i, hf=hf: (0, hf * (e_half // BE) + i)),
                pl.BlockSpec((d_edge, H), lambda i: (0, 0)),
                pl.BlockSpec((1, H), lambda i: (0, 0)),
                pl.BlockSpec((H, H), lambda i: (0, 0)),
                pl.BlockSpec((1, H), lambda i: (0, 0)),
                pl.BlockSpec((H, H), lambda i: (0, 0)),
                pl.BlockSpec((1, H), lambda i: (0, 0)),
            ],
            out_specs=pl.BlockSpec((2, BE // 4, 4 * HH), lambda i: (0, i, 0)),
            out_shape=jax.ShapeDtypeStruct((2, e_half // 4, 4 * HH),
                                           jnp.float32),
        )(ea_p, W1, b1.reshape(1, H), W2, b2.reshape(1, H), Wl,
          bl.reshape(1, H))

        aggrs.append(sc(x2f, srcp2, dstp2, e2))

    assert n_acc % BN == 0
    hi0 = n_acc // BN
    out = pl.pallas_call(
        _node_body,
        grid=(pl.cdiv(n, BN),),
        in_specs=[
            pl.BlockSpec((H, BN), lambda i: (0, i)),
            pl.BlockSpec((BN, HH), lambda i: (i, 0)),
            pl.BlockSpec((BN, HH), lambda i: (i + hi0, 0)),
            pl.BlockSpec((BN, HH), lambda i: (i, 0)),
            pl.BlockSpec((BN, HH), lambda i: (i + hi0, 0)),
            pl.BlockSpec((H, H), lambda i: (0, 0)),
            pl.BlockSpec((1, H), lambda i: (0, 0)),
            pl.BlockSpec((H, H), lambda i: (0, 0)),
            pl.BlockSpec((1, H), lambda i: (0, 0)),
            pl.BlockSpec((H, 3), lambda i: (0, 0)),
            pl.BlockSpec((1, 3), lambda i: (0, 0)),
        ],
        out_specs=pl.BlockSpec((BN, 3), lambda i: (i, 0)),
        out_shape=jax.ShapeDtypeStruct((n, 3), jnp.float32),
    )(x.T, aggrs[0], aggrs[0], aggrs[1], aggrs[1], Wn1, bn1.reshape(1, H),
      Wn2, bn2.reshape(1, H), Wo, bo.reshape(1, 3))

    return out.reshape(-1, 9, 3)
